# baseline (speedup 1.0000x reference)
import functools

import jax
import jax.numpy as jnp
from jax import lax
from jax.experimental import pallas as pl
from jax.experimental.pallas import tpu as pltpu
from jax.experimental.pallas import tpu_sc as plsc

N = 10000
E = 320000
D = 128
DE = 4

NC = 2
NS = 16
NW = NC * NS
L = 16

DEP = 16
NPT = 320
NPAD = NW * NPT
CAP = 16384
CHA = 2000
KB = 128

_NEG_INF = float("-inf")


_ROWS_BLK = 2000


def _leaky(x):
    return jnp.where(x >= 0, x, jnp.float32(0.01) * x)


def _mm2_body(x_ref, wa_ref, wb_ref, ya_ref, yb_ref):
    x = x_ref[...]
    ya_ref[...] = jnp.dot(x, wa_ref[...], preferred_element_type=jnp.float32)
    yb_ref[...] = jnp.dot(x, wb_ref[...], preferred_element_type=jnp.float32)


def _tc_mm2(x, wa, wb):
    grid = (N // _ROWS_BLK,)
    return pl.pallas_call(
        _mm2_body,
        grid=grid,
        in_specs=[
            pl.BlockSpec((_ROWS_BLK, D), lambda i: (i, 0)),
            pl.BlockSpec((D, D), lambda i: (0, 0)),
            pl.BlockSpec((D, D), lambda i: (0, 0)),
        ],
        out_specs=[
            pl.BlockSpec((_ROWS_BLK, D), lambda i: (i, 0)),
            pl.BlockSpec((_ROWS_BLK, D), lambda i: (i, 0)),
        ],
        out_shape=[
            jax.ShapeDtypeStruct((N, D), jnp.float32),
            jax.ShapeDtypeStruct((N, D), jnp.float32),
        ],
    )(x, wa, wb)


def _mid_body(agg_ref, r_ref, wa_ref, wb_ref, ya_ref, yb_ref):
    a = agg_ref[...]
    c = _leaky(jnp.where(jnp.isfinite(a), a, jnp.float32(0.0)) + r_ref[...])
    ya_ref[...] = jnp.dot(c, wa_ref[...], preferred_element_type=jnp.float32)
    yb_ref[...] = jnp.dot(c, wb_ref[...], preferred_element_type=jnp.float32)


def _tc_mid(agg, root, wa, wb):
    grid = (N // _ROWS_BLK,)
    return pl.pallas_call(
        _mid_body,
        grid=grid,
        in_specs=[
            pl.BlockSpec((_ROWS_BLK, D), lambda i: (i, 0)),
            pl.BlockSpec((_ROWS_BLK, D), lambda i: (i, 0)),
            pl.BlockSpec((D, D), lambda i: (0, 0)),
            pl.BlockSpec((D, D), lambda i: (0, 0)),
        ],
        out_specs=[
            pl.BlockSpec((_ROWS_BLK, D), lambda i: (i, 0)),
            pl.BlockSpec((_ROWS_BLK, D), lambda i: (i, 0)),
        ],
        out_shape=[
            jax.ShapeDtypeStruct((N, D), jnp.float32),
            jax.ShapeDtypeStruct((N, D), jnp.float32),
        ],
    )(agg, root, wa, wb)


def _fin_body(agg_ref, r_ref, o_ref):
    a = agg_ref[...]
    o_ref[...] = _leaky(jnp.where(jnp.isfinite(a), a, jnp.float32(0.0)) + r_ref[...])


def _tc_fin(agg, root):
    grid = (N // _ROWS_BLK,)
    return pl.pallas_call(
        _fin_body,
        grid=grid,
        in_specs=[
            pl.BlockSpec((_ROWS_BLK, D), lambda i: (i, 0)),
            pl.BlockSpec((_ROWS_BLK, D), lambda i: (i, 0)),
        ],
        out_specs=pl.BlockSpec((_ROWS_BLK, D), lambda i: (i, 0)),
        out_shape=jax.ShapeDtypeStruct((N, D), jnp.float32),
    )(agg, root)


_MESH = plsc.VectorSubcoreMesh(
    core_axis_name="c", subcore_axis_name="s", num_cores=NC, num_subcores=NS
)


def _init_lists(src_l, dst_l, eid_l, acc):
    zeros = jnp.zeros((L,), jnp.int32)
    dummy = jnp.full((L,), NPT, jnp.int32)

    def fill_lists(i, _):
        src_l[pl.ds(i * L, L)] = zeros
        dst_l[pl.ds(i * L, L)] = dummy
        eid_l[pl.ds(i * L, L)] = zeros
        return 0

    lax.fori_loop(0, CAP // L, fill_lists, 0)

    ninf = jnp.full((L,), _NEG_INF, jnp.float32)

    def fill_acc(i, _):
        for r in range(D // L):
            acc[i, pl.ds(r * L, L)] = ninf
        return 0

    lax.fori_loop(0, NPT + 1, fill_acc, 0)


def _phase_a(wid, src_hbm, dst_hbm, src_c, dst_c, src_l, dst_l, eid_l):
    lo = wid * NPT
    hi = lo + NPT
    iota = lax.iota(jnp.int32, L)
    ones = jnp.ones((L,), jnp.int32)
    zeros = jnp.zeros((L,), jnp.int32)

    def chunk_body(cb, cnt_v):
        base = cb * CHA
        pltpu.sync_copy(src_hbm.at[pl.ds(base, CHA)], src_c)
        pltpu.sync_copy(dst_hbm.at[pl.ds(base, CHA)], dst_c)

        def grp_body(g, cnt_v):
            dv = dst_c[pl.ds(g * L, L)]
            sv = src_c[pl.ds(g * L, L)]
            m = (dv >= lo) & (dv < hi)
            incl = plsc.cumsum(jnp.where(m, ones, zeros))
            tgt = incl + cnt_v - 1
            m2 = m & (tgt < CAP)
            plsc.store_scatter(src_l, [tgt], sv, mask=m2)
            plsc.store_scatter(dst_l, [tgt], dv - lo, mask=m2)
            plsc.store_scatter(eid_l, [tgt], iota + (base + g * L), mask=m2)
            return cnt_v + plsc.all_reduce_population_count(m)

        return lax.fori_loop(0, CHA // L, grp_body, cnt_v)

    cnt_v = lax.fori_loop(0, E // CHA, chunk_body, jnp.zeros((L,), jnp.int32))
    return cnt_v


def _phase_b(cnt, src_l, dst_l, eid_l, ea_hbm, y_hbm, we_v, be_v, acc,
             ybuf, eabuf, sem_y, sem_e):
    nchunks = (cnt + (KB - 1)) // KB

    def chunk_body(c, _):
        cbase = c * KB
        cp_y = pltpu.async_copy(y_hbm.at[src_l.at[pl.ds(cbase, KB)]], ybuf, sem_y)
        cp_e = pltpu.async_copy(ea_hbm.at[eid_l.at[pl.ds(cbase, KB)]], eabuf, sem_e)
        cp_y.wait()
        cp_e.wait()

        ws = [[we_v[k, pl.ds(r * L, L)] for r in range(D // L)] for k in range(DE)]
        bs = [be_v[pl.ds(r * L, L)] for r in range(D // L)]
        iota = lax.iota(jnp.int32, L)

        def grp_body(g, _):
            dv = dst_l[pl.ds(cbase + g * L, L)]
            rowi = g * L + iota
            eav = [
                plsc.load_gather(eabuf, [rowi, jnp.full((L,), k, jnp.int32)])
                for k in range(DE)
            ]
            for j in range(L):
                d = dv[j]
                e = g * L + j
                eaj = [jnp.full((L,), eav[k][j]) for k in range(DE)]
                for r in range(D // L):
                    gt = bs[r]
                    for k in range(DE):
                        gt = gt + eaj[k] * ws[k][r]
                    gate = jnp.float32(1.0) / (jnp.float32(1.0) + jnp.exp(-gt))
                    y = ybuf[e, pl.ds(r * L, L)]
                    msg = y * gate
                    acc[d, pl.ds(r * L, L)] = jnp.maximum(
                        acc[d, pl.ds(r * L, L)], msg)
            return 0

        lax.fori_loop(0, KB // L, grp_body, 0)
        return 0

    lax.fori_loop(0, nchunks, chunk_body, 0)


def _sc_scratch():
    return [
        pltpu.VMEM((NPT + 1, D), jnp.float32),
        pltpu.VMEM((CAP,), jnp.int32),
        pltpu.VMEM((CAP,), jnp.int32),
        pltpu.VMEM((CAP,), jnp.int32),
        pltpu.VMEM((CHA,), jnp.int32),
        pltpu.VMEM((CHA,), jnp.int32),
        pltpu.VMEM((KB, D), jnp.float32),
        pltpu.VMEM((KB, DEP), jnp.float32),
        pltpu.VMEM((DE, D), jnp.float32),
        pltpu.VMEM((D,), jnp.float32),
        pltpu.VMEM((L,), jnp.int32),
        pltpu.SemaphoreType.DMA,
        pltpu.SemaphoreType.DMA,
    ]


@functools.partial(
    pl.kernel,
    out_type=(
        jax.ShapeDtypeStruct((NPAD, D), jnp.float32),
        jax.ShapeDtypeStruct((NW, CAP), jnp.int32),
        jax.ShapeDtypeStruct((NW, CAP), jnp.int32),
        jax.ShapeDtypeStruct((NW, CAP), jnp.int32),
        jax.ShapeDtypeStruct((NW, L), jnp.int32),
    ),
    mesh=_MESH,
    scratch_types=_sc_scratch(),
    compiler_params=pltpu.CompilerParams(
        needs_layout_passes=False, use_tc_tiling_on_sc=False),
)
def _sc_layer1(src_hbm, dst_hbm, ea_hbm, we_hbm, be_hbm, y_hbm,
               agg_out, srcl_out, dstl_out, eidl_out, cnt_out,
               acc, src_l, dst_l, eid_l, src_c, dst_c, ybuf, eabuf,
               we_v, be_v, cnt_s, sem_y, sem_e):
    wid = lax.axis_index("s") * NC + lax.axis_index("c")
    _init_lists(src_l, dst_l, eid_l, acc)
    pltpu.sync_copy(we_hbm, we_v)
    pltpu.sync_copy(be_hbm, be_v)

    cnt_v = _phase_a(wid, src_hbm, dst_hbm, src_c, dst_c, src_l, dst_l, eid_l)
    cnt = jnp.max(cnt_v)

    _phase_b(cnt, src_l, dst_l, eid_l, ea_hbm, y_hbm, we_v, be_v, acc,
             ybuf, eabuf, sem_y, sem_e)

    pltpu.sync_copy(acc.at[pl.ds(0, NPT)], agg_out.at[pl.ds(wid * NPT, NPT)])
    pltpu.sync_copy(src_l, srcl_out.at[wid])
    pltpu.sync_copy(dst_l, dstl_out.at[wid])
    pltpu.sync_copy(eid_l, eidl_out.at[wid])
    cnt_s[...] = cnt_v
    pltpu.sync_copy(cnt_s, cnt_out.at[wid])


@functools.partial(
    pl.kernel,
    out_type=jax.ShapeDtypeStruct((NPAD, D), jnp.float32),
    mesh=_MESH,
    scratch_types=_sc_scratch(),
    compiler_params=pltpu.CompilerParams(
        needs_layout_passes=False, use_tc_tiling_on_sc=False),
)
def _sc_layer2(srcl_hbm, dstl_hbm, eidl_hbm, cnt_hbm, ea_hbm, we_hbm, be_hbm,
               y_hbm, agg_out,
               acc, src_l, dst_l, eid_l, src_c, dst_c, ybuf, eabuf,
               we_v, be_v, cnt_s, sem_y, sem_e):
    wid = lax.axis_index("s") * NC + lax.axis_index("c")

    ninf = jnp.full((L,), _NEG_INF, jnp.float32)

    def fill_acc(i, _):
        for r in range(D // L):
            acc[i, pl.ds(r * L, L)] = ninf
        return 0

    lax.fori_loop(0, NPT + 1, fill_acc, 0)

    pltpu.sync_copy(we_hbm, we_v)
    pltpu.sync_copy(be_hbm, be_v)
    pltpu.sync_copy(srcl_hbm.at[wid], src_l)
    pltpu.sync_copy(dstl_hbm.at[wid], dst_l)
    pltpu.sync_copy(eidl_hbm.at[wid], eid_l)
    pltpu.sync_copy(cnt_hbm.at[wid], cnt_s)
    cnt = jnp.max(cnt_s[...])

    _phase_b(cnt, src_l, dst_l, eid_l, ea_hbm, y_hbm, we_v, be_v, acc,
             ybuf, eabuf, sem_y, sem_e)

    pltpu.sync_copy(acc.at[pl.ds(0, NPT)], agg_out.at[pl.ds(wid * NPT, NPT)])


def kernel(X, edge_index, edge_attr, cent_n_id,
           W_msg1, W_edge1, b_edge1, W_root1,
           W_msg2, W_edge2, b_edge2, W_root2):
    del cent_n_id
    src = edge_index[0]
    dst = edge_index[1]
    ea_pad = jnp.pad(edge_attr, ((0, 0), (0, DEP - DE)))

    y1, r1 = _tc_mm2(X, W_msg1, W_root1)
    agg1, srcl, dstl, eidl, cnts = _sc_layer1(
        src, dst, ea_pad, W_edge1, b_edge1, y1)
    y2, r2 = _tc_mid(agg1[:N], r1, W_msg2, W_root2)
    agg2 = _sc_layer2(srcl, dstl, eidl, cnts, ea_pad, W_edge2, b_edge2, y2)
    return _tc_fin(agg2[:N], r2)
